# baseline (speedup 1.0000x reference)
import functools

import jax
import jax.numpy as jnp
from jax import lax
from jax.experimental import pallas as pl
from jax.experimental.pallas import tpu as pltpu
from jax.experimental.pallas import tpu_sc as plsc

VOCAB_SIZE = 1000000
D = 32
B = 16384

NC = 2
NS = 16
L = 16
NW = NC * NS
BPW = B // NW

_mesh = plsc.VectorSubcoreMesh(
    core_axis_name="c", subcore_axis_name="s", num_cores=NC, num_subcores=NS
)


@functools.partial(
    pl.kernel,
    mesh=_mesh,
    out_type=jax.ShapeDtypeStruct((B,), jnp.float32),
    scratch_types=[
        pltpu.VMEM((BPW,), jnp.int32),
        pltpu.VMEM((BPW,), jnp.int32),
        pltpu.VMEM((BPW, D), jnp.float32),
        pltpu.VMEM((BPW, D), jnp.float32),
        pltpu.VMEM((BPW,), jnp.float32),
        pltpu.VMEM((BPW,), jnp.float32),
        pltpu.VMEM((BPW,), jnp.float32),
        pltpu.SemaphoreType.DMA,
        pltpu.SemaphoreType.DMA,
        pltpu.SemaphoreType.DMA,
        pltpu.SemaphoreType.DMA,
    ],
    compiler_params=pltpu.CompilerParams(
        needs_layout_passes=False, use_tc_tiling_on_sc=False
    ),
)
def _glove_sc(target_hbm, context_hbm, wt_hbm, wc_hbm, ba_hbm, bb_hbm,
              out_hbm, idx_t, idx_c, rows_t, rows_c, bias_a, bias_b, obuf,
              sem_t, sem_c, sem_a, sem_b):
    wid = lax.axis_index("s") * NC + lax.axis_index("c")
    base = wid * BPW

    pltpu.sync_copy(target_hbm.at[pl.ds(base, BPW)], idx_t)
    pltpu.sync_copy(context_hbm.at[pl.ds(base, BPW)], idx_c)

    cp_t = pltpu.async_copy(wt_hbm.at[idx_t], rows_t, sem_t)
    cp_c = pltpu.async_copy(wc_hbm.at[idx_c], rows_c, sem_c)
    cp_a = pltpu.async_copy(ba_hbm.at[idx_t], bias_a, sem_a)
    cp_b = pltpu.async_copy(bb_hbm.at[idx_c], bias_b, sem_b)
    cp_t.wait()
    cp_c.wait()
    cp_a.wait()
    cp_b.wait()

    def body(g, carry):
        row0 = g * L
        acc = bias_a[pl.ds(row0, L)] + bias_b[pl.ds(row0, L)]
        rows = row0 + lax.iota(jnp.int32, L)
        for j in range(D):
            col = jnp.full((L,), j, jnp.int32)
            t = plsc.load_gather(rows_t, [rows, col])
            c = plsc.load_gather(rows_c, [rows, col])
            acc = acc + t * c
        obuf[pl.ds(row0, L)] = acc
        return carry

    lax.fori_loop(0, BPW // L, body, 0)
    pltpu.sync_copy(obuf, out_hbm.at[pl.ds(base, BPW)])


def kernel(target, context, W_target, W_context, b_a, b_b):
    return _glove_sc(target, context, W_target, W_context,
                     b_a.reshape(-1), b_b.reshape(-1))
